# baseline (speedup 1.0000x reference)
import functools

import jax
import jax.numpy as jnp
from jax.experimental import pallas as pl
from jax.experimental.pallas import tpu as pltpu

_K = 2
_GC = 0.7978845608028654


def _ffn_body(e_ref, x_ref, wu_ref, wd_ref, bu_ref, bd_ref, rw_ref, out_ref):
    j = pl.program_id(0)
    prev = e_ref[jnp.maximum(j - 1, 0)]
    first = (j == 0) | (e_ref[j] != prev)

    @pl.when(first)
    def _process_expert():
        h = jax.lax.dot_general(x_ref[...], wu_ref[0], (((1,), (1,)), ((), ())),
                                preferred_element_type=jnp.float32)
        h = h + bu_ref[0]
        inner = _GC * (h + 0.044715 * (h * h * h))
        h = h * (1.0 / (1.0 + jnp.exp(-2.0 * inner)))
        o = jax.lax.dot_general(h, wd_ref[0], (((1,), (1,)), ((), ())),
                                preferred_element_type=jnp.float32)
        o = (o + bd_ref[0]) * rw_ref[0, :, 0:1]

        @pl.when(j == 0)
        def _init():
            out_ref[...] = o

        @pl.when(j > 0)
        def _acc():
            out_ref[...] = out_ref[...] + o


@functools.partial(jax.jit, static_argnames=())
def kernel(x, W_router, W_up, W_down, b_up, b_down):
    b, s, d = x.shape
    e, u, _ = W_up.shape
    k = _K
    bs = b * s
    x2 = x.reshape(bs, d)

    logits = x2 @ W_router
    top_logits, indices = jax.lax.top_k(logits, k)
    rw = jax.nn.softmax(top_logits, axis=-1)
    flat_e = indices.reshape(-1).astype(jnp.int32)
    flat_t = (jnp.arange(bs * k, dtype=jnp.int32) // k)
    flat_w = rw.reshape(-1)
    e_s = jnp.sort(flat_e)
    rw3 = jnp.zeros((e, bs, 128), jnp.float32).at[flat_e, flat_t, 0].add(flat_w)

    npairs = bs * k

    grid_spec = pltpu.PrefetchScalarGridSpec(
        num_scalar_prefetch=1,
        grid=(npairs,),
        in_specs=[
            pl.BlockSpec((bs, d), lambda j, er: (0, 0)),
            pl.BlockSpec((1, u, d), lambda j, er: (er[j], 0, 0)),
            pl.BlockSpec((1, d, u), lambda j, er: (er[j], 0, 0)),
            pl.BlockSpec((1, 1, u), lambda j, er: (er[j], 0, 0)),
            pl.BlockSpec((1, 1, d), lambda j, er: (er[j], 0, 0)),
            pl.BlockSpec((1, bs, 128), lambda j, er: (er[j], 0, 0)),
        ],
        out_specs=pl.BlockSpec((bs, d), lambda j, er: (0, 0)),
    )

    out = pl.pallas_call(
        _ffn_body,
        grid_spec=grid_spec,
        out_shape=jax.ShapeDtypeStruct((bs, d), jnp.float32),
        compiler_params=pltpu.CompilerParams(
            dimension_semantics=("arbitrary",),
        ),
    )(e_s, x2, W_up, W_down,
      b_up.reshape(e, 1, u), b_down.reshape(e, 1, d), rw3)
    return out.reshape(b, s, d)
